# Initial kernel scaffold; baseline (speedup 1.0000x reference)
import jax
import jax.numpy as jnp
from jax.experimental import pallas as pl


def kernel(x_type, edge_index, batch, emb, W1, b1, W2, b2):
    raise NotImplementedError("write your pallas kernel here")


# speedup vs baseline: 20.9957x; 20.9957x over previous
import jax
import jax.numpy as jnp
from jax import lax
from jax.experimental import pallas as pl
from jax.experimental.pallas import tpu as pltpu
from jax.experimental.pallas import tpu_sc as plsc

N = 10000
E = 320000
NUM_TYPES = 512
EMB = 128
HID = 128
NUM_GRAPHS = 64

NC = 2
NS = 16
NW = NC * NS
EB = 128
NB_E = 80
CB = 8
EP = NW * NB_E * EB
NP = 10240
RPT = NP // NS
XB = 40
NB_X = (NP // NW) // XB
DEGW = 128

_f32 = jnp.float32


def _worker_id():
  c = lax.axis_index("c")
  s = lax.axis_index("s")
  return s * NC + c, c, s


def _sc_deg_gather_body(xt_hbm, dst_hbm, emb_hbm, ones_hbm, zeros16_hbm,
                        deg_out, x_out,
                        dst_idx, xt_idx, ones_v, rows_v, sem, acc):
  wid, c, s = _worker_id()
  pltpu.sync_copy(dst_hbm.at[pl.ds(wid * NB_E, NB_E)], dst_idx)
  pltpu.sync_copy(xt_hbm.at[wid], xt_idx)
  pltpu.sync_copy(ones_hbm, ones_v)
  pltpu.sync_copy(zeros16_hbm.at[pl.ds(s * RPT, RPT)],
                  acc.at[pl.ds(s * RPT, RPT)])
  plsc.subcore_barrier()

  def deg_body(b):
    pltpu.sync_copy(ones_v, acc.at[dst_idx.at[b]], add=True)
  pl.loop(0, NB_E)(deg_body)

  def gather_body(j):
    pltpu.async_copy(emb_hbm.at[xt_idx.at[j]], rows_v, sem).wait()
    pltpu.sync_copy(rows_v,
                    x_out.at[pl.ds(wid * (NB_X * XB) + j * XB, XB)])
  pl.loop(0, NB_X)(gather_body)

  plsc.subcore_barrier()
  pltpu.sync_copy(acc.at[pl.ds(s * RPT, RPT)],
                  deg_out.at[c, pl.ds(s * RPT, RPT)])


def _sc_conv_body(g_hbm, src_hbm, dst_hbm, zeros_hbm,
                  acc_out,
                  src_c, dst_c, rows0, rows1, sem0, sem1, acc):
  wid, c, s = _worker_id()
  pltpu.sync_copy(zeros_hbm.at[pl.ds(s * RPT, RPT)],
                  acc.at[pl.ds(s * RPT, RPT)])
  plsc.subcore_barrier()

  bufs = ((rows0, sem0), (rows1, sem1))

  def chunk(j):
    base = wid * NB_E + j * CB
    pltpu.sync_copy(src_hbm.at[pl.ds(base, CB)], src_c)
    pltpu.sync_copy(dst_hbm.at[pl.ds(base, CB)], dst_c)
    pltpu.async_copy(g_hbm.at[src_c.at[0]], rows0, sem0)
    for t in range(CB):
      buf, sem = bufs[t % 2]
      if t + 1 < CB:
        nbuf, nsem = bufs[(t + 1) % 2]
        pltpu.async_copy(g_hbm.at[src_c.at[t + 1]], nbuf, nsem)
      pltpu.make_async_copy(g_hbm.at[src_c.at[t]], buf, sem).wait()
      pltpu.sync_copy(buf, acc.at[dst_c.at[t]], add=True)

  pl.loop(0, NB_E // CB)(chunk)

  plsc.subcore_barrier()
  pltpu.sync_copy(acc.at[pl.ds(s * RPT, RPT)],
                  acc_out.at[c, pl.ds(s * RPT, RPT)])


def _make_sc_kernels():
  mesh = plsc.VectorSubcoreMesh(core_axis_name="c", subcore_axis_name="s")
  deg_gather = pl.kernel(
      _sc_deg_gather_body,
      out_type=(
          jax.ShapeDtypeStruct((NC, NP, DEGW), _f32),
          jax.ShapeDtypeStruct((NP, EMB), _f32),
      ),
      mesh=mesh,
      scratch_types=[
          pltpu.VMEM((NB_E, EB), jnp.int32),
          pltpu.VMEM((NB_X, XB), jnp.int32),
          pltpu.VMEM((EB, DEGW), _f32),
          pltpu.VMEM((XB, EMB), _f32),
          pltpu.SemaphoreType.DMA,
          pltpu.VMEM_SHARED((NP, DEGW), _f32),
      ],
      name="gnn_sc_deg_gather",
  )
  conv = pl.kernel(
      _sc_conv_body,
      out_type=jax.ShapeDtypeStruct((NC, NP, HID), _f32),
      mesh=mesh,
      scratch_types=[
          pltpu.VMEM((CB, EB), jnp.int32),
          pltpu.VMEM((CB, EB), jnp.int32),
          pltpu.VMEM((EB, HID), _f32),
          pltpu.VMEM((EB, HID), _f32),
          pltpu.SemaphoreType.DMA,
          pltpu.SemaphoreType.DMA,
          pltpu.VMEM_SHARED((NP, HID), _f32),
      ],
      name="gnn_sc_conv",
  )
  return deg_gather, conv


_ROWS_B = 1024
_GRID = NP // _ROWS_B


def _tc_prep_body(x_ref, d0_ref, d1_ref, w1_ref, g1_ref, dinv_ref):
  pid = pl.program_id(0)
  deg = d0_ref[:, 0:1] + d1_ref[:, 0:1] + 1.0
  dinv = lax.rsqrt(deg)
  dinvb = jnp.broadcast_to(dinv, (_ROWS_B, HID))
  h = jnp.dot(x_ref[...], w1_ref[...], preferred_element_type=_f32)
  row = pid * _ROWS_B + lax.broadcasted_iota(jnp.int32, (_ROWS_B, HID), 0)
  g1_ref[...] = jnp.where(row < N, h * dinvb, 0.0)
  dinv_ref[...] = dinvb


def _tc_mid_body(a_ref, g1_ref, dinv_ref, b1_ref, w2_ref, g2_ref):
  pid = pl.program_id(0)
  dinv = dinv_ref[...]
  z1 = dinv * (a_ref[0] + a_ref[1] + g1_ref[...]) + b1_ref[...]
  z1 = jnp.maximum(z1, 0.0)
  h2 = jnp.dot(z1, w2_ref[...], preferred_element_type=_f32)
  row = pid * _ROWS_B + lax.broadcasted_iota(jnp.int32, (_ROWS_B, HID), 0)
  g2_ref[...] = jnp.where(row < N, h2 * dinv, 0.0)


def _tc_pool_body(a_ref, g2_ref, dinv_ref, b2_ref, bt_ref, out_ref):
  pid = pl.program_id(0)
  z2 = dinv_ref[...] * (a_ref[0] + a_ref[1] + g2_ref[...]) + b2_ref[...]
  bt = bt_ref[0]
  gid = lax.broadcasted_iota(jnp.int32, (NUM_GRAPHS, _ROWS_B), 0)
  onehot = (gid == jnp.broadcast_to(bt, (NUM_GRAPHS, _ROWS_B))).astype(_f32)
  contrib = jnp.dot(onehot, z2, preferred_element_type=_f32)

  @pl.when(pid == 0)
  def _():
    out_ref[...] = contrib

  @pl.when(pid > 0)
  def _():
    out_ref[...] = out_ref[...] + contrib


def _row_spec(width):
  return pl.BlockSpec((_ROWS_B, width), lambda i: (i, 0))


def _acc_spec():
  return pl.BlockSpec((NC, _ROWS_B, HID), lambda i: (0, i, 0))


def _const_spec(shape):
  nd = len(shape)
  return pl.BlockSpec(shape, lambda i: (0,) * nd)


def _tc_prep(x, d0, d1, w1):
  return pl.pallas_call(
      _tc_prep_body,
      grid=(_GRID,),
      in_specs=[_row_spec(EMB), _row_spec(DEGW), _row_spec(DEGW),
                _const_spec((EMB, HID))],
      out_specs=[_row_spec(HID), _row_spec(HID)],
      out_shape=[jax.ShapeDtypeStruct((NP, HID), _f32),
                 jax.ShapeDtypeStruct((NP, HID), _f32)],
  )(x, d0, d1, w1)


def _tc_mid(a, g1, dinv, b1, w2):
  return pl.pallas_call(
      _tc_mid_body,
      grid=(_GRID,),
      in_specs=[_acc_spec(), _row_spec(HID), _row_spec(HID),
                _const_spec((1, HID)), _const_spec((HID, HID))],
      out_specs=_row_spec(HID),
      out_shape=jax.ShapeDtypeStruct((NP, HID), _f32),
  )(a, g1, dinv, b1, w2)


def _tc_pool(a, g2, dinv, b2, batch3):
  return pl.pallas_call(
      _tc_pool_body,
      grid=(_GRID,),
      in_specs=[_acc_spec(), _row_spec(HID), _row_spec(HID),
                _const_spec((1, HID)),
                pl.BlockSpec((1, 1, _ROWS_B), lambda i: (i, 0, 0))],
      out_specs=_const_spec((NUM_GRAPHS, HID)),
      out_shape=jax.ShapeDtypeStruct((NUM_GRAPHS, HID), _f32),
  )(a, g2, dinv, b2, batch3)


@jax.jit
def kernel(x_type, edge_index, batch, emb, W1, b1, W2, b2):
  i32 = jnp.int32
  src = edge_index[0].astype(i32)
  dst = edge_index[1].astype(i32)

  pad = N + (jnp.arange(EP - E, dtype=i32) % (NP - N))
  src2 = jnp.concatenate([src, pad]).reshape(NW * NB_E, EB)
  dst2 = jnp.concatenate([dst, pad]).reshape(NW * NB_E, EB)
  xt = jnp.concatenate(
      [x_type.astype(i32), jnp.zeros((NP - N,), i32)]).reshape(NW, NB_X, XB)
  batch3 = jnp.concatenate(
      [batch.astype(i32),
       jnp.full((NP - N,), NUM_GRAPHS, i32)]).reshape(_GRID, 1, _ROWS_B)

  ones128 = jnp.ones((EB, DEGW), _f32)
  zeros128 = jnp.zeros((NP, HID), _f32)

  deg_gather, conv = _make_sc_kernels()

  degp, x = deg_gather(xt, dst2, emb, ones128, zeros128)
  g1, dinv = _tc_prep(x, degp[0], degp[1], W1)

  acc1 = conv(g1, src2, dst2, zeros128)
  g2 = _tc_mid(acc1, g1, dinv, b1.reshape(1, HID), W2)

  acc2 = conv(g2, src2, dst2, zeros128)
  out = _tc_pool(acc2, g2, dinv, b2.reshape(1, HID), batch3)
  return out
